# baseline (speedup 1.0000x reference)
import jax
import jax.numpy as jnp
from jax import lax
from jax.experimental import pallas as pl
from jax.experimental.pallas import tpu as pltpu
from jax.experimental.pallas import tpu_sc as plsc

N_NUM = 13
N_CAT = 26
CARD = 1000
D = 128
B = 4096
N_TOK = N_NUM + N_CAT

try:
    _INFO = plsc.get_sparse_core_info()
    _NC = _INFO.num_cores
    _NS = _INFO.num_subcores
except Exception:
    _NC = 2
    _NS = 16
_NW = _NC * _NS
_RPW = B // _NW
_NV = D // 16


def _sc_body(xnum_hbm, xcatt_hbm, w_hbm, nb_hbm, tab_hbm, cb_hbm, out_hbm,
             w_v, nb_v, cb_v, xnum_v, idx0, idx1, gat0, gat1, ob0, ob1,
             sg0, sg1, so0, so1):
    wid = lax.axis_index("s") * _NC + lax.axis_index("c")
    base = wid * _RPW

    idxs = [idx0, idx1]
    gats = [gat0, gat1]
    obs = [ob0, ob1]
    sgs = [sg0, sg1]
    sos = [so0, so1]

    pltpu.sync_copy(w_hbm, w_v)
    pltpu.sync_copy(nb_hbm, nb_v)
    pltpu.sync_copy(cb_hbm, cb_v)
    pltpu.sync_copy(xnum_hbm.at[pl.ds(base, _RPW)], xnum_v)

    def out_slab(t):
        return out_hbm.at[t, pl.ds(base, _RPW)]

    def load_codes_and_gather(f):
        g = f % 2
        pltpu.sync_copy(xcatt_hbm.at[f, pl.ds(base, _RPW)], idxs[g])
        for j in range(_RPW // 16):
            s = pl.ds(j * 16, 16)
            idxs[g][s] = idxs[g][s] + f * CARD
        pltpu.async_copy(tab_hbm.at[idxs[g]], gats[g], sgs[g])

    load_codes_and_gather(0)
    load_codes_and_gather(1)

    for t in range(N_NUM):
        o = t % 2
        if t >= 2:
            pltpu.make_async_copy(obs[o], out_slab(t - 2), sos[o]).wait()

        def num_fill(b, carry, t=t, o=o):
            xs = xnum_v[b][t]
            for i in range(_NV):
                s = pl.ds(i * 16, 16)
                obs[o][b, s] = xs * w_v[t, s] + nb_v[t, s]
            return carry

        lax.fori_loop(0, _RPW, num_fill, 0)
        pltpu.async_copy(obs[o], out_slab(t), sos[o])

    for f in range(N_CAT):
        g = f % 2
        o = (N_NUM + f) % 2
        pltpu.make_async_copy(tab_hbm.at[idxs[g]], gats[g], sgs[g]).wait()

        pltpu.make_async_copy(obs[o], out_slab(N_NUM + f - 2), sos[o]).wait()

        def cat_fill(b, carry, f=f, g=g, o=o):
            for i in range(_NV):
                s = pl.ds(i * 16, 16)
                obs[o][b, s] = gats[g][b, s] + cb_v[f, s]
            return carry

        lax.fori_loop(0, _RPW, cat_fill, 0)
        pltpu.async_copy(obs[o], out_slab(N_NUM + f), sos[o])
        if f + 2 < N_CAT:
            load_codes_and_gather(f + 2)

    pltpu.make_async_copy(obs[(N_TOK - 2) % 2],
                          out_slab(N_TOK - 2), sos[(N_TOK - 2) % 2]).wait()
    pltpu.make_async_copy(obs[(N_TOK - 1) % 2],
                          out_slab(N_TOK - 1), sos[(N_TOK - 1) % 2]).wait()


@jax.jit
def _run(x_num_pad, x_cat_t, w_pad, nb_pad, cat_table, cat_bias):
    mesh = plsc.VectorSubcoreMesh(core_axis_name="c", subcore_axis_name="s")
    sc = pl.kernel(
        _sc_body,
        mesh=mesh,
        out_type=jax.ShapeDtypeStruct((N_TOK, B, D), jnp.float32),
        scratch_types=[
            pltpu.VMEM((16, D), jnp.float32),
            pltpu.VMEM((16, D), jnp.float32),
            pltpu.VMEM((N_CAT, D), jnp.float32),
            pltpu.VMEM((_RPW, 16), jnp.float32),
            pltpu.VMEM((_RPW,), jnp.int32),
            pltpu.VMEM((_RPW,), jnp.int32),
            pltpu.VMEM((_RPW, D), jnp.float32),
            pltpu.VMEM((_RPW, D), jnp.float32),
            pltpu.VMEM((_RPW, D), jnp.float32),
            pltpu.VMEM((_RPW, D), jnp.float32),
            pltpu.SemaphoreType.DMA,
            pltpu.SemaphoreType.DMA,
            pltpu.SemaphoreType.DMA,
            pltpu.SemaphoreType.DMA,
        ],
    )
    out_tm = sc(x_num_pad, x_cat_t, w_pad, nb_pad, cat_table, cat_bias)
    return jnp.transpose(out_tm, (1, 0, 2))


def kernel(x_num, x_cat, num_weight, num_bias, cat_table, cat_bias):
    x_num_pad = jnp.pad(x_num, ((0, 0), (0, 16 - N_NUM)))
    x_cat_t = x_cat.astype(jnp.int32).T
    w_pad = jnp.pad(num_weight, ((0, 16 - N_NUM), (0, 0)))
    nb_pad = jnp.pad(num_bias, ((0, 16 - N_NUM), (0, 0)))
    return _run(x_num_pad, x_cat_t, w_pad, nb_pad, cat_table, cat_bias)


# speedup vs baseline: 1.4657x; 1.4657x over previous
import jax
import jax.numpy as jnp
from jax import lax
from jax.experimental import pallas as pl
from jax.experimental.pallas import tpu as pltpu
from jax.experimental.pallas import tpu_sc as plsc

N_NUM = 13
N_CAT = 26
CARD = 1000
D = 128
B = 4096
N_TOK = N_NUM + N_CAT

try:
    _INFO = plsc.get_sparse_core_info()
    _NC = _INFO.num_cores
    _NS = _INFO.num_subcores
except Exception:
    _NC = 2
    _NS = 16
_NW = _NC * _NS
_RPW = B // _NW
_NV = D // 16
_NB = 6
_LEAD = 4


def _fold_body(tab_ref, bias_ref, out_ref):
    out_ref[...] = tab_ref[...] + bias_ref[0]


def _fold_table(cat_table, cat_bias):
    return pl.pallas_call(
        _fold_body,
        grid=(N_CAT,),
        in_specs=[
            pl.BlockSpec((CARD, D), lambda i: (i, 0)),
            pl.BlockSpec((1, 1, D), lambda i: (i, 0, 0)),
        ],
        out_specs=pl.BlockSpec((CARD, D), lambda i: (i, 0)),
        out_shape=jax.ShapeDtypeStruct((N_CAT * CARD, D), jnp.float32),
    )(cat_table, cat_bias[:, None, :])


def _sc_body(xnum_hbm, xcatt_hbm, w_hbm, nb_hbm, tab_hbm, out_hbm,
             w_v, nb_v, xnum_v,
             ix0, ix1, ix2, ix3, ix4, ix5,
             sb0, sb1, sb2, sb3, sb4, sb5,
             sg0, sg1, sg2, sg3, sg4, sg5,
             so0, so1, so2, so3, so4, so5):
    wid = lax.axis_index("s") * _NC + lax.axis_index("c")
    base = wid * _RPW

    ixs = [ix0, ix1, ix2, ix3, ix4, ix5]
    sbs = [sb0, sb1, sb2, sb3, sb4, sb5]
    sgs = [sg0, sg1, sg2, sg3, sg4, sg5]
    sos = [so0, so1, so2, so3, so4, so5]

    pltpu.sync_copy(w_hbm, w_v)
    pltpu.sync_copy(nb_hbm, nb_v)
    pltpu.sync_copy(xnum_hbm.at[pl.ds(base, _RPW)], xnum_v)

    def out_slab(t):
        return out_hbm.at[t, pl.ds(base, _RPW)]

    def start_gather(f, b):
        pltpu.sync_copy(xcatt_hbm.at[f, pl.ds(base, _RPW)], ixs[b])
        for j in range(_RPW // 16):
            s = pl.ds(j * 16, 16)
            ixs[b][s] = ixs[b][s] + f * CARD
        pltpu.async_copy(tab_hbm.at[ixs[b]], sbs[b], sgs[b])

    def wait_out(t, b):
        pltpu.make_async_copy(sbs[b], out_slab(t), sos[b]).wait()

    for f in range(_LEAD):
        start_gather(f, f)

    for t in range(N_NUM):
        b = 4 + t % 2
        if t >= 2:
            wait_out(t - 2, b)

        def num_fill(r, carry, t=t, b=b):
            xs = xnum_v[r][t]
            for i in range(_NV):
                s = pl.ds(i * 16, 16)
                sbs[b][r, s] = xs * w_v[t, s] + nb_v[t, s]
            return carry

        lax.fori_loop(0, _RPW, num_fill, 0)
        pltpu.async_copy(sbs[b], out_slab(t), sos[b])

    for f in range(N_CAT):
        b = f % _NB
        pltpu.make_async_copy(tab_hbm.at[ixs[b]], sbs[b], sgs[b]).wait()
        pltpu.async_copy(sbs[b], out_slab(N_NUM + f), sos[b])
        nf = f + _LEAD
        if nf < N_CAT:
            nb_ = nf % _NB
            prev_t = N_NUM + f - 2 if f >= 2 else (12 if nb_ == 4 else 11)
            wait_out(prev_t, nb_)
            start_gather(nf, nb_)

    for f in range(N_CAT - _LEAD, N_CAT):
        wait_out(N_NUM + f, f % _NB)


@jax.jit
def _run(x_num_pad, x_cat_t, w_pad, nb_pad, cat_table, cat_bias):
    folded = _fold_table(cat_table, cat_bias)
    mesh = plsc.VectorSubcoreMesh(core_axis_name="c", subcore_axis_name="s")
    sc = pl.kernel(
        _sc_body,
        mesh=mesh,
        out_type=jax.ShapeDtypeStruct((N_TOK, B, D), jnp.float32),
        scratch_types=(
            [pltpu.VMEM((16, D), jnp.float32),
             pltpu.VMEM((16, D), jnp.float32),
             pltpu.VMEM((_RPW, 16), jnp.float32)]
            + [pltpu.VMEM((_RPW,), jnp.int32) for _ in range(_NB)]
            + [pltpu.VMEM((_RPW, D), jnp.float32) for _ in range(_NB)]
            + [pltpu.SemaphoreType.DMA for _ in range(2 * _NB)]
        ),
    )
    out_tm = sc(x_num_pad, x_cat_t, w_pad, nb_pad, folded)
    return jnp.transpose(out_tm, (1, 0, 2))


def kernel(x_num, x_cat, num_weight, num_bias, cat_table, cat_bias):
    x_num_pad = jnp.pad(x_num, ((0, 0), (0, 16 - N_NUM)))
    x_cat_t = x_cat.astype(jnp.int32).T
    w_pad = jnp.pad(num_weight, ((0, 16 - N_NUM), (0, 0)))
    nb_pad = jnp.pad(num_bias, ((0, 16 - N_NUM), (0, 0)))
    return _run(x_num_pad, x_cat_t, w_pad, nb_pad, cat_table, cat_bias)


# speedup vs baseline: 1.8566x; 1.2667x over previous
import jax
import jax.numpy as jnp
from jax import lax
from jax.experimental import pallas as pl
from jax.experimental.pallas import tpu as pltpu
from jax.experimental.pallas import tpu_sc as plsc

N_NUM = 13
N_CAT = 26
CARD = 1000
D = 128
B = 4096
N_TOK = N_NUM + N_CAT

try:
    _INFO = plsc.get_sparse_core_info()
    _NC = _INFO.num_cores
    _NS = _INFO.num_subcores
except Exception:
    _NC = 2
    _NS = 16
_NW = _NC * _NS
_CH = 256
_CPT = (N_CAT * B) // _CH // _NW
_NBUF = 3


def _fold_body(tab_ref, bias_ref, out_ref):
    out_ref[...] = tab_ref[...] + bias_ref[0]


def _fold_table(cat_table, cat_bias):
    return pl.pallas_call(
        _fold_body,
        grid=(N_CAT,),
        in_specs=[
            pl.BlockSpec((CARD, D), lambda i: (i, 0)),
            pl.BlockSpec((1, 1, D), lambda i: (i, 0, 0)),
        ],
        out_specs=pl.BlockSpec((CARD, D), lambda i: (i, 0)),
        out_shape=jax.ShapeDtypeStruct((N_CAT * CARD, D), jnp.float32),
    )(cat_table, cat_bias[:, None, :])


def _sc_body(xcatt_hbm, tab_hbm, out_hbm,
             ix0, ix1, ix2, sb0, sb1, sb2,
             sg0, sg1, sg2, so0, so1, so2):
    wid = lax.axis_index("s") * _NC + lax.axis_index("c")
    k0 = wid * _CPT

    ixs = [ix0, ix1, ix2]
    sbs = [sb0, sb1, sb2]
    sgs = [sg0, sg1, sg2]
    sos = [so0, so1, so2]

    def chunk_coords(j):
        k = k0 + j
        f = lax.div(k, jnp.int32(B // _CH))
        b0 = lax.rem(k, jnp.int32(B // _CH)) * _CH
        return f, b0

    def start_gather(j, b):
        f, b0 = chunk_coords(j)
        pltpu.sync_copy(xcatt_hbm.at[f, pl.ds(b0, _CH)], ixs[b])
        for v in range(_CH // 16):
            s = pl.ds(v * 16, 16)
            ixs[b][s] = ixs[b][s] + f * CARD
        pltpu.async_copy(tab_hbm.at[ixs[b]], sbs[b], sgs[b])

    def issue_out(j, b):
        f, b0 = chunk_coords(j)
        pltpu.async_copy(sbs[b], out_hbm.at[N_NUM + f, pl.ds(b0, _CH)],
                         sos[b])

    def wait_gather(b):
        pltpu.make_async_copy(tab_hbm.at[ixs[b]], sbs[b], sgs[b]).wait()

    def wait_out(b):
        pltpu.make_async_copy(sbs[b], out_hbm.at[0, pl.ds(0, _CH)],
                              sos[b]).wait()

    start_gather(0, 0)
    start_gather(1, 1)
    for j in range(_CPT):
        b = j % _NBUF
        wait_gather(b)
        issue_out(j, b)
        nj = j + 2
        if nj < _CPT:
            nb_ = nj % _NBUF
            if nj >= _NBUF:
                wait_out(nb_)
            start_gather(nj, nb_)
    for j in range(_CPT - _NBUF, _CPT):
        wait_out(j % _NBUF)


def _sc_gather(x_cat_t, folded):
    mesh = plsc.VectorSubcoreMesh(core_axis_name="c", subcore_axis_name="s")
    sc = pl.kernel(
        _sc_body,
        mesh=mesh,
        out_type=jax.ShapeDtypeStruct((N_TOK, B, D), jnp.float32),
        scratch_types=(
            [pltpu.VMEM((_CH,), jnp.int32) for _ in range(_NBUF)]
            + [pltpu.VMEM((_CH, D), jnp.float32) for _ in range(_NBUF)]
            + [pltpu.SemaphoreType.DMA for _ in range(2 * _NBUF)]
        ),
    )
    return sc(x_cat_t, folded)


_BB = 512


def _num_body(out_in_ref, xn_ref, w_ref, nb_ref, out_ref):
    del out_in_ref
    x = xn_ref[0, 0]
    w = w_ref[0, 0]
    nb = nb_ref[0, 0]
    out_ref[0] = x[:, None] * w[None, :] + nb[None, :]


def _fill_num(catout, x_num_t3, w3, nb3):
    return pl.pallas_call(
        _num_body,
        grid=(N_NUM, B // _BB),
        in_specs=[
            pl.BlockSpec(memory_space=pl.ANY),
            pl.BlockSpec((1, 1, _BB), lambda t, i: (t, 0, i)),
            pl.BlockSpec((1, 1, D), lambda t, i: (t, 0, 0)),
            pl.BlockSpec((1, 1, D), lambda t, i: (t, 0, 0)),
        ],
        out_specs=pl.BlockSpec((1, _BB, D), lambda t, i: (t, i, 0)),
        out_shape=jax.ShapeDtypeStruct((N_TOK, B, D), jnp.float32),
        input_output_aliases={0: 0},
    )(catout, x_num_t3, w3, nb3)


@jax.jit
def _run(x_num_t3, x_cat_t, w3, nb3, cat_table, cat_bias):
    folded = _fold_table(cat_table, cat_bias)
    catout = _sc_gather(x_cat_t, folded)
    out_tm = _fill_num(catout, x_num_t3, w3, nb3)
    return jnp.transpose(out_tm, (1, 0, 2))


def kernel(x_num, x_cat, num_weight, num_bias, cat_table, cat_bias):
    x_num_t3 = x_num.T[:, None, :]
    x_cat_t = x_cat.astype(jnp.int32).T
    w3 = num_weight[:, None, :]
    nb3 = num_bias[:, None, :]
    return _run(x_num_t3, x_cat_t, w3, nb3, cat_table, cat_bias)


# speedup vs baseline: 2.3116x; 1.2451x over previous
import jax
import jax.numpy as jnp
from jax import lax
from jax.experimental import pallas as pl
from jax.experimental.pallas import tpu as pltpu
from jax.experimental.pallas import tpu_sc as plsc

N_NUM = 13
N_CAT = 26
CARD = 1000
D = 128
B = 4096
N_TOK = N_NUM + N_CAT

try:
    _INFO = plsc.get_sparse_core_info()
    _NC = _INFO.num_cores
    _NS = _INFO.num_subcores
except Exception:
    _NC = 2
    _NS = 16
_NW = _NC * _NS
_CH = 256
_CPT = (N_CAT * B) // _CH // _NW
_NBUF = 3


def _fold_body(tab_ref, bias_ref, out_ref):
    out_ref[:CARD] = tab_ref[:CARD] + bias_ref[0]
    out_ref[CARD:] = tab_ref[CARD:] + bias_ref[1]


def _fold_table(cat_table, cat_bias):
    return pl.pallas_call(
        _fold_body,
        grid=(N_CAT // 2,),
        in_specs=[
            pl.BlockSpec((2 * CARD, D), lambda i: (i, 0)),
            pl.BlockSpec((2, 1, D), lambda i: (i, 0, 0)),
        ],
        out_specs=pl.BlockSpec((2 * CARD, D), lambda i: (i, 0)),
        out_shape=jax.ShapeDtypeStruct((N_CAT * CARD, D), jnp.float32),
    )(cat_table, cat_bias[:, None, :])


def _sc_body(xcatt_hbm, tab_hbm, out_hbm,
             ix0, ix1, ix2, sb0, sb1, sb2,
             sg0, sg1, sg2, so0, so1, so2):
    wid = lax.axis_index("s") * _NC + lax.axis_index("c")
    k0 = wid * _CPT

    ixs = [ix0, ix1, ix2]
    sbs = [sb0, sb1, sb2]
    sgs = [sg0, sg1, sg2]
    sos = [so0, so1, so2]

    def chunk_coords(j):
        k = k0 + j
        f = lax.div(k, jnp.int32(B // _CH))
        b0 = lax.rem(k, jnp.int32(B // _CH)) * _CH
        return f, b0

    def start_gather(j, b):
        f, b0 = chunk_coords(j)
        pltpu.sync_copy(xcatt_hbm.at[f, pl.ds(b0, _CH)], ixs[b])
        for v in range(_CH // 16):
            s = pl.ds(v * 16, 16)
            ixs[b][s] = ixs[b][s] + f * CARD
        pltpu.async_copy(tab_hbm.at[ixs[b]], sbs[b], sgs[b])

    def issue_out(j, b):
        f, b0 = chunk_coords(j)
        pltpu.async_copy(sbs[b], out_hbm.at[N_NUM + f, pl.ds(b0, _CH)],
                         sos[b])

    def wait_gather(b):
        pltpu.make_async_copy(tab_hbm.at[ixs[b]], sbs[b], sgs[b]).wait()

    def wait_out(b):
        pltpu.make_async_copy(sbs[b], out_hbm.at[0, pl.ds(0, _CH)],
                              sos[b]).wait()

    start_gather(0, 0)
    start_gather(1, 1)
    for j in range(_CPT):
        b = j % _NBUF
        wait_gather(b)
        issue_out(j, b)
        nj = j + 2
        if nj < _CPT:
            nb_ = nj % _NBUF
            if nj >= _NBUF:
                wait_out(nb_)
            start_gather(nj, nb_)
    for j in range(_CPT - _NBUF, _CPT):
        wait_out(j % _NBUF)


def _sc_gather(x_cat_t, folded):
    mesh = plsc.VectorSubcoreMesh(core_axis_name="c", subcore_axis_name="s")
    sc = pl.kernel(
        _sc_body,
        mesh=mesh,
        out_type=jax.ShapeDtypeStruct((N_TOK, B, D), jnp.float32),
        scratch_types=(
            [pltpu.VMEM((_CH,), jnp.int32) for _ in range(_NBUF)]
            + [pltpu.VMEM((_CH, D), jnp.float32) for _ in range(_NBUF)]
            + [pltpu.SemaphoreType.DMA for _ in range(2 * _NBUF)]
        ),
    )
    return sc(x_cat_t, folded)


_BB = 1024


def _num_body(out_in_ref, xn_ref, w_ref, nb_ref, out_ref):
    del out_in_ref
    x = xn_ref[0, 0]
    w = w_ref[0, 0]
    nb = nb_ref[0, 0]
    prod = jax.lax.dot_general(
        x[:, None], w[None, :], (((1,), (0,)), ((), ())),
        preferred_element_type=jnp.float32)
    out_ref[0] = prod + nb[None, :]


def _fill_num(catout, x_num_t3, w3, nb3):
    return pl.pallas_call(
        _num_body,
        grid=(N_NUM, B // _BB),
        in_specs=[
            pl.BlockSpec(memory_space=pl.ANY),
            pl.BlockSpec((1, 1, _BB), lambda t, i: (t, 0, i)),
            pl.BlockSpec((1, 1, D), lambda t, i: (t, 0, 0)),
            pl.BlockSpec((1, 1, D), lambda t, i: (t, 0, 0)),
        ],
        out_specs=pl.BlockSpec((1, _BB, D), lambda t, i: (t, i, 0)),
        out_shape=jax.ShapeDtypeStruct((N_TOK, B, D), jnp.float32),
        input_output_aliases={0: 0},
    )(catout, x_num_t3, w3, nb3)


@jax.jit
def _run(x_num_t3, x_cat_t, w3, nb3, cat_table, cat_bias):
    folded = _fold_table(cat_table, cat_bias)
    catout = _sc_gather(x_cat_t, folded)
    out_tm = _fill_num(catout, x_num_t3, w3, nb3)
    return jnp.transpose(out_tm, (1, 0, 2))


def kernel(x_num, x_cat, num_weight, num_bias, cat_table, cat_bias):
    x_num_t3 = x_num.T[:, None, :]
    x_cat_t = x_cat.astype(jnp.int32).T
    w3 = num_weight[:, None, :]
    nb3 = num_bias[:, None, :]
    return _run(x_num_t3, x_cat_t, w3, nb3, cat_table, cat_bias)
